# baseline (speedup 1.0000x reference)
import functools
import numpy as np
import jax
import jax.numpy as jnp
from jax import lax
from jax.experimental import pallas as pl
from jax.experimental.pallas import tpu as pltpu
from jax.experimental.pallas import tpu_sc as plsc

_C, _D, _H, _W = 4, 128, 224, 224
_NS = 64
_NP = _C * _D
_NW = 32
_PPW = _NP // _NW
_L = 16

_AX = np.linspace(0, _D - 1, _NS).astype(np.int32)
_SG = np.linspace(0, _W - 1, _NS).astype(np.int32)
_CO = np.linspace(0, _H - 1, _NS).astype(np.int32)
assert all(int(_SG[s]) == (s * (_W - 1)) // (_NS - 1) for s in range(_NS))
assert all(int(_CO[s]) == (s * (_H - 1)) // (_NS - 1) for s in range(_NS))
assert all(int(_AX[s]) == (2 * s if s < 63 else 127) for s in range(_NS))


def _sc_body(vol4, ax4, sag4, cor4, plane_v, sag_v, cor_v):
    wid = lax.axis_index("s") * 2 + lax.axis_index("c")

    def do_plane(i, _):
        pd = wid * _PPW + i
        c = pd // _D
        d = pd % _D

        pltpu.sync_copy(vol4.at[c, d], plane_v)

        @plsc.parallel_loop(0, _NS, unroll=4)
        def do_s(s):
            w_s = lax.div(s * (_W - 1), _NS - 1)
            h_s = lax.div(s * (_H - 1), _NS - 1)
            for j in range(_H // _L):
                hvec = lax.iota(jnp.int32, _L) + (j * _L)
                wvec = jnp.full((_L,), w_s, jnp.int32)
                sag_v[s, pl.ds(j * _L, _L)] = plsc.load_gather(
                    plane_v, [hvec, wvec])
            for j in range(_W // _L):
                cor_v[s, pl.ds(j * _L, _L)] = plane_v[h_s, pl.ds(j * _L, _L)]

        pltpu.sync_copy(sag_v, sag4.at[c, :, d, :])
        pltpu.sync_copy(cor_v, cor4.at[c, :, d, :])

        is_ax = jnp.logical_or(
            jnp.logical_and(d % 2 == 0, d <= 124), d == 127)
        s_ax = jnp.where(d == 127, 63, d // 2)

        @pl.when(is_ax)
        def _do_ax():
            pltpu.sync_copy(plane_v, ax4.at[c, s_ax])

        return 0

    lax.fori_loop(0, _PPW, do_plane, 0)


@jax.jit
def kernel(volume):
    mesh = plsc.VectorSubcoreMesh(core_axis_name="c", subcore_axis_name="s")
    k = functools.partial(
        pl.kernel,
        mesh=mesh,
        out_type=[
            jax.ShapeDtypeStruct((_C, _NS, _H, _W), jnp.float32),
            jax.ShapeDtypeStruct((_C, _NS, _D, _H), jnp.float32),
            jax.ShapeDtypeStruct((_C, _NS, _D, _W), jnp.float32),
        ],
        scratch_types=[
            pltpu.VMEM((_H, _W), jnp.float32),
            pltpu.VMEM((_NS, _H), jnp.float32),
            pltpu.VMEM((_NS, _W), jnp.float32),
        ],
        compiler_params=pltpu.CompilerParams(needs_layout_passes=False),
    )(_sc_body)
    ax, sag, cor = k(volume)
    return (ax, sag, cor)


# speedup vs baseline: 1.5289x; 1.5289x over previous
import numpy as np
import jax
import jax.numpy as jnp
from jax.experimental import pallas as pl
from jax.experimental.pallas import tpu as pltpu

_C, _D, _H, _W = 4, 128, 224, 224
_NS = 64
_DBLK = 32
_NK = _D // _DBLK
_SBLK = _NS // _NK

_AX = np.linspace(0, _D - 1, _NS).astype(np.int32)
_SG = np.linspace(0, _W - 1, _NS).astype(np.int32)
_CO = np.linspace(0, _H - 1, _NS).astype(np.int32)

assert all(_AX[k * _SBLK + j] // _DBLK == k
           for k in range(_NK) for j in range(_SBLK))
_AX_LOCAL = _AX.reshape(_NK, _SBLK) - (np.arange(_NK) * _DBLK)[:, None]


def _onehot(idx, n):
    m = np.zeros((_NS, n), np.float32)
    m[np.arange(_NS), idx] = 1.0
    return jnp.asarray(m)


def _body(oh_co_ref, oh_sg_ref, vol_ref, ax_ref, sag_ref, cor_ref, tp_ref):
    k = pl.program_id(1)
    for p in range(_DBLK):
        tp_ref[p] = vol_ref[0, p].T
    for s in range(_NS):
        sag_ref[0, s, :, :] = tp_ref[:, int(_SG[s]), :]
    for s in range(_NS):
        cor_ref[0, s, :, :] = vol_ref[0, :, int(_CO[s]), :]
    for j in range(_SBLK):
        if np.all(_AX_LOCAL[:, j] == _AX_LOCAL[0, j]):
            ax_ref[0, j] = vol_ref[0, int(_AX_LOCAL[0, j])]
        else:
            lj = jnp.where(k == _NK - 1, int(_AX_LOCAL[-1, j]),
                           int(_AX_LOCAL[0, j]))
            ax_ref[0, j] = vol_ref[0, lj]


@jax.jit
def kernel(volume):
    oh_co = _onehot(_CO, _H)
    oh_sg = _onehot(_SG, _W)
    grid = (_C, _NK)
    out = pl.pallas_call(
        _body,
        grid=grid,
        in_specs=[
            pl.BlockSpec((_NS, _H), lambda c, k: (0, 0)),
            pl.BlockSpec((_NS, _W), lambda c, k: (0, 0)),
            pl.BlockSpec((1, _DBLK, _H, _W), lambda c, k: (c, k, 0, 0)),
        ],
        out_specs=[
            pl.BlockSpec((1, _SBLK, _H, _W), lambda c, k: (c, k, 0, 0)),
            pl.BlockSpec((1, _NS, _DBLK, _H), lambda c, k: (c, 0, k, 0)),
            pl.BlockSpec((1, _NS, _DBLK, _W), lambda c, k: (c, 0, k, 0)),
        ],
        out_shape=[
            jax.ShapeDtypeStruct((_C, _NS, _H, _W), jnp.float32),
            jax.ShapeDtypeStruct((_C, _NS, _D, _H), jnp.float32),
            jax.ShapeDtypeStruct((_C, _NS, _D, _W), jnp.float32),
        ],
        scratch_shapes=[pltpu.VMEM((_DBLK, _W, _H), jnp.float32)],
        compiler_params=pltpu.CompilerParams(
            dimension_semantics=("parallel", "parallel")),
    )(oh_co, oh_sg, volume)
    axial, sagittal, coronal = out
    return (axial, sagittal, coronal)
